# Initial kernel scaffold; baseline (speedup 1.0000x reference)
import jax
import jax.numpy as jnp
from jax.experimental import pallas as pl


def kernel(x0_max_values, x1_argmax_indices):
    raise NotImplementedError("write your pallas kernel here")


# speedup vs baseline: 7.7598x; 7.7598x over previous
import functools

import jax
import jax.numpy as jnp
from jax import lax
from jax.experimental import pallas as pl
from jax.experimental.pallas import tpu as pltpu
from jax.experimental.pallas import tpu_sc as plsc

B, H, W, C = 4, 112, 112, 192
IMG = H * W * C
OUT = IMG * 4
NTILES = 32
NPER = OUT // NTILES
BLK = 2048
NBLK = NPER // BLK
SUB = 512
NQ = 14
CHUNK = OUT // NQ
FR = 32
FU = FR * 16
CAP = FU * 640
BUFCAP = 64 * 16
ZW = 1024
WPW = CHUNK // 16
NROUNDS = 2 * NQ
ZOFF = 2 * BUFCAP
COFF = BUFCAP
MOFF = BUFCAP + 256

_mesh = plsc.VectorSubcoreMesh(core_axis_name="c", subcore_axis_name="s")


def _m8(x):
    return pl.multiple_of(x, 8)


@functools.partial(
    pl.kernel,
    mesh=_mesh,
    compiler_params=pltpu.CompilerParams(needs_layout_passes=False),
    out_type=[
        jax.ShapeDtypeStruct((B * OUT,), jnp.float32),
        jax.ShapeDtypeStruct((NTILES * NQ * CAP,), jnp.int32),
        jax.ShapeDtypeStruct((NTILES * NQ * CAP,), jnp.float32),
    ],
    scratch_types=[
        pltpu.VMEM((2 * BLK,), jnp.int32),
        pltpu.VMEM((2 * BLK,), jnp.float32),
        pltpu.VMEM((NQ * BUFCAP,), jnp.int32),
        pltpu.VMEM((NQ * BUFCAP,), jnp.float32),
        pltpu.VMEM((256,), jnp.int32),
        pltpu.VMEM((FU,), jnp.int32),
        pltpu.VMEM((FU,), jnp.float32),
        pltpu.SMEM((16,), jnp.int32),
        pltpu.SMEM((16,), jnp.int32),
        pltpu.VMEM_SHARED((CHUNK,), jnp.float32),
        pltpu.VMEM_SHARED((256,), jnp.int32),
        pltpu.SemaphoreType.DMA,
        pltpu.SemaphoreType.DMA,
        pltpu.SemaphoreType.DMA,
        pltpu.SemaphoreType.DMA,
        pltpu.SemaphoreType.DMA,
        pltpu.SemaphoreType.DMA,
    ],
)
def _unpool_sc(x0v, x1i, out, pidx, pval, inb_i, inb_v, bidx, bval, fillmat,
               stg_i, stg_v, fills, hfills, chunk, ctab,
               sA, sB, sF, sZ, sPA, sPB):
    c = lax.axis_index("c")
    s = lax.axis_index("s")
    g = c * 16 + s
    batch = c * 2 + (s >> 3)
    in_base = batch * IMG + (s & 7) * NPER

    zv = jnp.zeros((16,), jnp.float32)
    lanes = lax.iota(jnp.int32, 16)

    for q in range(NQ):
        hfills[q] = 0
    fills[15] = 0

    ziv = jnp.zeros((16,), jnp.int32)

    def _init(j, carry):
        bidx[pl.ds(j * 16, 16)] = ziv
        bval[pl.ds(j * 16, 16)] = jnp.zeros((16,), jnp.float32)
        return carry

    lax.fori_loop(0, NQ * BUFCAP // 16, _init, 0)
    for j in range(16):
        fillmat[pl.ds(j * 16, 16)] = ziv

    def _drain_flush():
        @pl.when(fills[15] > 0)
        def _w():
            pltpu.make_async_copy(pidx.at[pl.ds(0, FU)], stg_i, sF).wait()
            pltpu.make_async_copy(pval.at[pl.ds(0, FU)], stg_v, sF).wait()
            fills[15] = 0

    ones = jnp.ones((16,), jnp.int32)

    def _flush_ready():
        def _fl(q, carry3):
            frow = fillmat[pl.ds(q * 16, 16)]
            need = frow >= FR
            n = plsc.all_reduce_population_count(need)[0]

            @pl.when(n > 0)
            def _f0():
                _drain_flush()

                def _st(j, carry4):
                    o = j * 16
                    stg_i[pl.ds(o, 16)] = bidx[pl.ds(q * BUFCAP + o, 16)]
                    stg_v[pl.ds(o, 16)] = bval[pl.ds(q * BUFCAP + o, 16)]
                    return carry4

                lax.fori_loop(0, FU // 16, _st, 0)
                hf = hfills[q]
                pltpu.async_copy(
                    stg_i, pidx.at[pl.ds(_m8((g * NQ + q) * CAP + hf), FU)],
                    sF)
                pltpu.async_copy(
                    stg_v, pval.at[pl.ds(_m8((g * NQ + q) * CAP + hf), FU)],
                    sF)
                fills[15] = 1

                newf = jnp.maximum(frow - FR, 0)

                def _cp(r, carry4):
                    dst = q * BUFCAP + r * 16
                    src = dst + FU
                    bidx[pl.ds(dst, 16)] = bidx[pl.ds(src, 16)]
                    vrow = bval[pl.ds(src, 16)]
                    bval[pl.ds(dst, 16)] = jnp.where(newf > r, vrow, zv)
                    return carry4

                lax.fori_loop(0, FR, _cp, 0)
                fillmat[pl.ds(q * 16, 16)] = newf
                hfills[q] = hfills[q] + FU
            return carry3

        lax.fori_loop(0, NQ, _fl, 0)

    def _proc(base):
        for sub in range(BLK // SUB):
            def _vreg(j, carry2, sub=sub):
                o = base + sub * SUB + j * 16
                iv = inb_i[pl.ds(o, 16)]
                vv = inb_v[pl.ds(o, 16)]
                q = ((iv >> 15) * 1561) >> 15
                tloc = iv - q * CHUNK
                fidx = q * 16 + lanes
                f = plsc.load_gather(fillmat, [fidx])
                addr = q * BUFCAP + f * 16 + lanes
                plsc.store_scatter(bidx, [addr], tloc)
                plsc.store_scatter(bval, [addr], vv)
                plsc.addupdate_scatter(fillmat, [fidx], ones)
                return carry2

            lax.fori_loop(0, SUB // 16, _vreg, 0)
            _flush_ready()

    def _fire(i, base, sem):
        off = _m8(in_base + i * BLK)
        pltpu.async_copy(x1i.at[pl.ds(off, BLK)],
                         inb_i.at[pl.ds(base, BLK)], sem)
        pltpu.async_copy(x0v.at[pl.ds(off, BLK)],
                         inb_v.at[pl.ds(base, BLK)], sem)

    def _wait_in(i, base, sem):
        off = _m8(in_base + i * BLK)
        pltpu.make_async_copy(x1i.at[pl.ds(off, BLK)],
                              inb_i.at[pl.ds(base, BLK)], sem).wait()
        pltpu.make_async_copy(x0v.at[pl.ds(off, BLK)],
                              inb_v.at[pl.ds(base, BLK)], sem).wait()

    _fire(0, 0, sA)

    def _pair(k, carry):
        i0 = 2 * k
        _fire(i0 + 1, BLK, sB)
        _wait_in(i0, 0, sA)
        _proc(0)
        _fire(i0 + 2, 0, sA)
        _wait_in(i0 + 1, BLK, sB)
        _proc(BLK)
        return carry

    lax.fori_loop(0, (NBLK - 1) // 2, _pair, 0)
    _wait_in(NBLK - 1, 0, sA)
    _proc(0)
    _drain_flush()

    def _drain(q, cnts):
        frow = fillmat[pl.ds(q * 16, 16)]
        n = plsc.all_reduce_population_count(frow >= 1)[0]

        def _pad(r, carry):
            o = q * BUFCAP + r * 16
            vrow = bval[pl.ds(o, 16)]
            bval[pl.ds(o, 16)] = jnp.where(frow > r, vrow, zv)
            return carry

        lax.fori_loop(0, FR, _pad, 0)

        @pl.when(n > 0)
        def _d0():
            hf = hfills[q]
            pltpu.sync_copy(
                bidx.at[pl.ds(_m8(q * BUFCAP), FU)],
                pidx.at[pl.ds(_m8((g * NQ + q) * CAP + hf), FU)])
            pltpu.sync_copy(
                bval.at[pl.ds(_m8(q * BUFCAP), FU)],
                pval.at[pl.ds(_m8((g * NQ + q) * CAP + hf), FU)])

        cq = hfills[q] + jnp.where(n > 0, FU, 0)
        return jnp.where(lanes == q, jnp.full((16,), 1, jnp.int32) * cq, cnts)

    cnts = lax.fori_loop(0, NQ, _drain, jnp.zeros((16,), jnp.int32))
    bidx[pl.ds(MOFF, 16)] = cnts

    pltpu.sync_copy(bidx.at[pl.ds(_m8(MOFF), 16)],
                    ctab.at[pl.ds(_m8(s * 16), 16)])
    plsc.subcore_barrier()
    pltpu.sync_copy(ctab, bidx.at[pl.ds(_m8(COFF), 256)])

    def _zb(j, carry):
        bval[pl.ds(ZOFF + j * 16, 16)] = zv
        return carry

    lax.fori_loop(0, ZW // 16, _zb, 0)

    e = s >> 1
    hh = s & 1

    P2A = 3 * BUFCAP
    P2B = 4 * BUFCAP

    def _round(r, carry):
        b_rel = r // NQ
        q = r - b_rel * NQ

        def _zfire(k, carry2):
            pltpu.async_copy(
                bval.at[pl.ds(_m8(ZOFF), ZW)],
                chunk.at[pl.ds(_m8(s * WPW + k * ZW), ZW)], sZ)
            return carry2

        def _zdrain(k, carry2):
            pltpu.make_async_copy(
                bval.at[pl.ds(_m8(ZOFF), ZW)],
                chunk.at[pl.ds(_m8(s * WPW + k * ZW), ZW)], sZ).wait()
            return carry2

        lax.fori_loop(0, WPW // ZW, _zfire, 0)
        lax.fori_loop(0, WPW // ZW, _zdrain, 0)
        plsc.subcore_barrier()

        s_src = b_rel * 8 + e
        g_src = c * 16 + s_src
        crow = bidx[pl.ds(COFF + s_src * 16, 16)]
        cnt = jnp.sum(jnp.where(lanes == q, crow, 0))
        nb = cnt >> 9
        n0 = nb >> 1
        mine = jnp.where(hh == 0, n0, nb - n0)
        start = hh * n0
        rbase = (g_src * NQ + q) * CAP

        def _pfire(i, base, sem):
            o = _m8(rbase + (start + i) * FU)
            pltpu.async_copy(pidx.at[pl.ds(o, FU)],
                             bidx.at[pl.ds(base, FU)], sem)
            pltpu.async_copy(pval.at[pl.ds(o, FU)],
                             bval.at[pl.ds(base, FU)], sem)

        def _pwait(i, base, sem):
            o = _m8(rbase + (start + i) * FU)
            pltpu.make_async_copy(pidx.at[pl.ds(o, FU)],
                                  bidx.at[pl.ds(base, FU)], sem).wait()
            pltpu.make_async_copy(pval.at[pl.ds(o, FU)],
                                  bval.at[pl.ds(base, FU)], sem).wait()

        @pl.when(mine > 0)
        def _prime():
            _pfire(0, P2A, sPA)

        def _pair2(k, carry2):
            i0 = 2 * k

            @pl.when(i0 + 1 < mine)
            def _fb():
                _pfire(i0 + 1, P2B, sPB)

            _pwait(i0, P2A, sPA)
            pltpu.sync_copy(bval.at[pl.ds(P2A, FU)],
                            chunk.at[bidx.at[pl.ds(P2A, FU)]], add=True)

            @pl.when(i0 + 2 < mine)
            def _fa():
                _pfire(i0 + 2, P2A, sPA)

            @pl.when(i0 + 1 < mine)
            def _wb():
                _pwait(i0 + 1, P2B, sPB)
                pltpu.sync_copy(bval.at[pl.ds(P2B, FU)],
                                chunk.at[bidx.at[pl.ds(P2B, FU)]], add=True)
            return carry2

        lax.fori_loop(0, (mine + 1) >> 1, _pair2, 0)
        plsc.subcore_barrier()

        out_off = (c * 2 + b_rel) * OUT + q * CHUNK + s * WPW
        pltpu.sync_copy(chunk.at[pl.ds(_m8(s * WPW), WPW)],
                        out.at[pl.ds(_m8(out_off), WPW)])
        plsc.subcore_barrier()
        return carry

    lax.fori_loop(0, NROUNDS, _round, 0)


def kernel(x0_max_values, x1_argmax_indices):
    x0f = x0_max_values.reshape(-1)
    x1f = x1_argmax_indices.reshape(-1).astype(jnp.int32)
    out, _, _ = _unpool_sc(x0f, x1f)
    return out.reshape(B, 2 * H, 2 * W, C)


# speedup vs baseline: 10.7112x; 1.3803x over previous
import functools

import jax
import jax.numpy as jnp
from jax import lax
from jax.experimental import pallas as pl
from jax.experimental.pallas import tpu as pltpu
from jax.experimental.pallas import tpu_sc as plsc

B, H, W, C = 4, 112, 112, 192
IMG = H * W * C
OUT = IMG * 4
NTILES = 32
NPER = OUT // NTILES
BLK = 2048
NBLK = NPER // BLK
SUB = 512
NQ = 14
CHUNK = OUT // NQ
FU = 512
CAP = FU * 590
BUFCAP = 1040
ZW = 1024
WPW = CHUNK // 16
NROUNDS = 2 * NQ
ZOFF = 2 * BUFCAP
COFF = BUFCAP
MOFF = BUFCAP + 256

_mesh = plsc.VectorSubcoreMesh(core_axis_name="c", subcore_axis_name="s")


def _m8(x):
    return pl.multiple_of(x, 8)


@functools.partial(
    pl.kernel,
    mesh=_mesh,
    compiler_params=pltpu.CompilerParams(needs_layout_passes=False),
    out_type=[
        jax.ShapeDtypeStruct((B * OUT,), jnp.float32),
        jax.ShapeDtypeStruct((NTILES * NQ * CAP,), jnp.int32),
        jax.ShapeDtypeStruct((NTILES * NQ * CAP,), jnp.float32),
    ],
    scratch_types=[
        pltpu.VMEM((2 * BLK,), jnp.int32),
        pltpu.VMEM((2 * BLK,), jnp.float32),
        pltpu.VMEM((NQ * BUFCAP,), jnp.int32),
        pltpu.VMEM((NQ * BUFCAP,), jnp.float32),
        pltpu.VMEM((FU,), jnp.int32),
        pltpu.VMEM((FU,), jnp.float32),
        pltpu.SMEM((16,), jnp.int32),
        pltpu.SMEM((16,), jnp.int32),
        pltpu.VMEM_SHARED((CHUNK,), jnp.float32),
        pltpu.VMEM_SHARED((256,), jnp.int32),
        pltpu.SemaphoreType.DMA,
        pltpu.SemaphoreType.DMA,
        pltpu.SemaphoreType.DMA,
        pltpu.SemaphoreType.DMA,
        pltpu.SemaphoreType.DMA,
        pltpu.SemaphoreType.DMA,
    ],
)
def _unpool_sc(x0v, x1i, out, pidx, pval, inb_i, inb_v, bidx, bval,
               stg_i, stg_v, fills, hfills, chunk, ctab,
               sA, sB, sF, sZ, sPA, sPB):
    c = lax.axis_index("c")
    s = lax.axis_index("s")
    g = c * 16 + s
    batch = c * 2 + (s >> 3)
    in_base = batch * IMG + (s & 7) * NPER

    zv = jnp.zeros((16,), jnp.float32)
    lanes = lax.iota(jnp.int32, 16)

    for q in range(NQ):
        fills[q] = 0
        hfills[q] = 0
    fills[15] = 0

    def _drain_flush():
        @pl.when(fills[15] > 0)
        def _w():
            pltpu.make_async_copy(pidx.at[pl.ds(0, FU)], stg_i, sF).wait()
            pltpu.make_async_copy(pval.at[pl.ds(0, FU)], stg_v, sF).wait()
            fills[15] = 0

    def _flush_ready():
        def _fl(q, carry3):
            @pl.when(fills[q] >= FU)
            def _f0():
                _drain_flush()

                def _st(j, carry4):
                    o = j * 16
                    stg_i[pl.ds(o, 16)] = bidx[pl.ds(q * BUFCAP + o, 16)]
                    stg_v[pl.ds(o, 16)] = bval[pl.ds(q * BUFCAP + o, 16)]
                    return carry4

                lax.fori_loop(0, FU // 16, _st, 0)
                hf = hfills[q]
                pltpu.async_copy(
                    stg_i, pidx.at[pl.ds(_m8((g * NQ + q) * CAP + hf), FU)],
                    sF)
                pltpu.async_copy(
                    stg_v, pval.at[pl.ds(_m8((g * NQ + q) * CAP + hf), FU)],
                    sF)
                fills[15] = 1

                def _cp(j, carry4):
                    o = j * 16
                    bidx[pl.ds(q * BUFCAP + o, 16)] = (
                        bidx[pl.ds(q * BUFCAP + FU + o, 16)])
                    bval[pl.ds(q * BUFCAP + o, 16)] = (
                        bval[pl.ds(q * BUFCAP + FU + o, 16)])
                    return carry4

                lax.fori_loop(0, FU // 16 + 1, _cp, 0)
                fills[q] = fills[q] - FU
                hfills[q] = hfills[q] + FU
            return carry3

        lax.fori_loop(0, NQ, _fl, 0)

    def _proc(base):
        for sub in range(BLK // SUB):
            def _vreg(j, carry2, sub=sub):
                o = base + sub * SUB + j * 16
                iv = inb_i[pl.ds(o, 16)]
                vv = inb_v[pl.ds(o, 16)]
                lt_prev = None
                for q in range(NQ):
                    if q < NQ - 1:
                        ltq = iv < (q + 1) * CHUNK
                        m = ltq if q == 0 else jnp.logical_and(
                            ltq, jnp.logical_not(lt_prev))
                    else:
                        m = jnp.logical_not(lt_prev)
                        ltq = None
                    fq = fills[q]
                    plsc.store_compressed(
                        bidx.at[pl.ds(q * BUFCAP + fq, 16)],
                        iv - q * CHUNK, mask=m)
                    plsc.store_compressed(
                        bval.at[pl.ds(q * BUFCAP + fq, 16)], vv, mask=m)
                    fills[q] = fq + plsc.all_reduce_population_count(m)[0]
                    lt_prev = ltq
                return carry2

            lax.fori_loop(0, SUB // 16, _vreg, 0)
            _flush_ready()

    def _fire(i, base, sem):
        off = _m8(in_base + i * BLK)
        pltpu.async_copy(x1i.at[pl.ds(off, BLK)],
                         inb_i.at[pl.ds(base, BLK)], sem)
        pltpu.async_copy(x0v.at[pl.ds(off, BLK)],
                         inb_v.at[pl.ds(base, BLK)], sem)

    def _wait_in(i, base, sem):
        off = _m8(in_base + i * BLK)
        pltpu.make_async_copy(x1i.at[pl.ds(off, BLK)],
                              inb_i.at[pl.ds(base, BLK)], sem).wait()
        pltpu.make_async_copy(x0v.at[pl.ds(off, BLK)],
                              inb_v.at[pl.ds(base, BLK)], sem).wait()

    _fire(0, 0, sA)

    def _pair(k, carry):
        i0 = 2 * k
        _fire(i0 + 1, BLK, sB)
        _wait_in(i0, 0, sA)
        _proc(0)
        _fire(i0 + 2, 0, sA)
        _wait_in(i0 + 1, BLK, sB)
        _proc(BLK)
        return carry

    lax.fori_loop(0, (NBLK - 1) // 2, _pair, 0)
    _wait_in(NBLK - 1, 0, sA)
    _proc(0)
    _drain_flush()

    def _drain(q, cnts):
        f = fills[q]
        base16 = q * BUFCAP + ((f >> 4) << 4)
        keep = lanes < (f & 15)
        cur_i = bidx[pl.ds(base16, 16)]
        cur_v = bval[pl.ds(base16, 16)]
        bidx[pl.ds(base16, 16)] = jnp.where(keep, cur_i, lanes)
        bval[pl.ds(base16, 16)] = jnp.where(keep, cur_v, zv)

        def _pad(j, carry):
            o = base16 + 16 + j * 16
            bidx[pl.ds(o, 16)] = lanes
            bval[pl.ds(o, 16)] = zv
            return carry

        lax.fori_loop(0, FU // 16, _pad, 0)

        @pl.when(f > 0)
        def _d0():
            hf = hfills[q]
            pltpu.sync_copy(
                bidx.at[pl.ds(_m8(q * BUFCAP), FU)],
                pidx.at[pl.ds(_m8((g * NQ + q) * CAP + hf), FU)])
            pltpu.sync_copy(
                bval.at[pl.ds(_m8(q * BUFCAP), FU)],
                pval.at[pl.ds(_m8((g * NQ + q) * CAP + hf), FU)])

        cq = hfills[q] + jnp.where(f > 0, FU, 0)
        return jnp.where(lanes == q, jnp.full((16,), 1, jnp.int32) * cq, cnts)

    cnts = lax.fori_loop(0, NQ, _drain, jnp.zeros((16,), jnp.int32))
    bidx[pl.ds(MOFF, 16)] = cnts

    pltpu.sync_copy(bidx.at[pl.ds(_m8(MOFF), 16)],
                    ctab.at[pl.ds(_m8(s * 16), 16)])
    plsc.subcore_barrier()
    pltpu.sync_copy(ctab, bidx.at[pl.ds(_m8(COFF), 256)])

    def _zb(j, carry):
        bval[pl.ds(ZOFF + j * 16, 16)] = zv
        return carry

    lax.fori_loop(0, ZW // 16, _zb, 0)

    e = s >> 1
    hh = s & 1

    P2A = 3 * BUFCAP
    P2B = 4 * BUFCAP

    def _round(r, carry):
        b_rel = r // NQ
        q = r - b_rel * NQ

        def _zfire(k, carry2):
            pltpu.async_copy(
                bval.at[pl.ds(_m8(ZOFF), ZW)],
                chunk.at[pl.ds(_m8(s * WPW + k * ZW), ZW)], sZ)
            return carry2

        def _zdrain(k, carry2):
            pltpu.make_async_copy(
                bval.at[pl.ds(_m8(ZOFF), ZW)],
                chunk.at[pl.ds(_m8(s * WPW + k * ZW), ZW)], sZ).wait()
            return carry2

        lax.fori_loop(0, WPW // ZW, _zfire, 0)
        lax.fori_loop(0, WPW // ZW, _zdrain, 0)
        plsc.subcore_barrier()

        s_src = b_rel * 8 + e
        g_src = c * 16 + s_src
        crow = bidx[pl.ds(COFF + s_src * 16, 16)]
        cnt = jnp.sum(jnp.where(lanes == q, crow, 0))
        nb = cnt >> 9
        n0 = nb >> 1
        mine = jnp.where(hh == 0, n0, nb - n0)
        start = hh * n0
        rbase = (g_src * NQ + q) * CAP

        def _pfire(i, base, sem):
            o = _m8(rbase + (start + i) * FU)
            pltpu.async_copy(pidx.at[pl.ds(o, FU)],
                             bidx.at[pl.ds(base, FU)], sem)
            pltpu.async_copy(pval.at[pl.ds(o, FU)],
                             bval.at[pl.ds(base, FU)], sem)

        def _pwait(i, base, sem):
            o = _m8(rbase + (start + i) * FU)
            pltpu.make_async_copy(pidx.at[pl.ds(o, FU)],
                                  bidx.at[pl.ds(base, FU)], sem).wait()
            pltpu.make_async_copy(pval.at[pl.ds(o, FU)],
                                  bval.at[pl.ds(base, FU)], sem).wait()

        @pl.when(mine > 0)
        def _prime():
            _pfire(0, P2A, sPA)

        def _pair2(k, carry2):
            i0 = 2 * k

            @pl.when(i0 + 1 < mine)
            def _fb():
                _pfire(i0 + 1, P2B, sPB)

            _pwait(i0, P2A, sPA)
            pltpu.sync_copy(bval.at[pl.ds(P2A, FU)],
                            chunk.at[bidx.at[pl.ds(P2A, FU)]], add=True)

            @pl.when(i0 + 2 < mine)
            def _fa():
                _pfire(i0 + 2, P2A, sPA)

            @pl.when(i0 + 1 < mine)
            def _wb():
                _pwait(i0 + 1, P2B, sPB)
                pltpu.sync_copy(bval.at[pl.ds(P2B, FU)],
                                chunk.at[bidx.at[pl.ds(P2B, FU)]], add=True)
            return carry2

        lax.fori_loop(0, (mine + 1) >> 1, _pair2, 0)
        plsc.subcore_barrier()

        out_off = (c * 2 + b_rel) * OUT + q * CHUNK + s * WPW
        pltpu.sync_copy(chunk.at[pl.ds(_m8(s * WPW), WPW)],
                        out.at[pl.ds(_m8(out_off), WPW)])
        plsc.subcore_barrier()
        return carry

    lax.fori_loop(0, NROUNDS, _round, 0)


def kernel(x0_max_values, x1_argmax_indices):
    x0f = x0_max_values.reshape(-1)
    x1f = x1_argmax_indices.reshape(-1).astype(jnp.int32)
    out, _, _ = _unpool_sc(x0f, x1f)
    return out.reshape(B, 2 * H, 2 * W, C)
